# baseline (speedup 1.0000x reference)
import functools

import jax
import jax.numpy as jnp
from jax import lax
from jax.experimental import pallas as pl
from jax.experimental.pallas import tpu as pltpu
from jax.experimental.pallas import tpu_sc as plsc

N_NODES = 10000
N_EDGES = 320000
DIM = 128

NUM_CORES = 2
NUM_SUBCORES = 16
CHUNK = 56
GROUP = 8
CPW0 = 184
CPW1 = 184
TOT_CHUNKS = NUM_SUBCORES * (CPW0 + CPW1)
E_PAD = TOT_CHUNKS * CHUNK
N_PAD = 10112
ROWS_PER_TILE = N_PAD // NUM_SUBCORES
NROW = 2
NIDX = 4


def _matmul_body(x_ref, w_ref, o_ref):
    o_ref[...] = jnp.dot(x_ref[...], w_ref[...],
                         preferred_element_type=jnp.float32)


def _support_matmul(x, W):
    blk = 400
    grid = N_NODES // blk
    return pl.pallas_call(
        _matmul_body,
        grid=(grid,),
        in_specs=[
            pl.BlockSpec((blk, DIM), lambda i: (i, 0)),
            pl.BlockSpec((DIM, DIM), lambda i: (0, 0)),
        ],
        out_specs=pl.BlockSpec((blk, DIM), lambda i: (i, 0)),
        out_shape=jax.ShapeDtypeStruct((N_NODES, DIM), jnp.float32),
    )(x, W)


def _combine_body(p_ref, b_ref, o_ref):
    o_ref[...] = p_ref[0] + p_ref[1] + b_ref[...]


def _combine(partials, b2d):
    blk = 400
    grid = N_NODES // blk
    return pl.pallas_call(
        _combine_body,
        grid=(grid,),
        in_specs=[
            pl.BlockSpec((2, blk, DIM), lambda i: (0, i, 0)),
            pl.BlockSpec((1, DIM), lambda i: (0, 0)),
        ],
        out_specs=pl.BlockSpec((blk, DIM), lambda i: (i, 0)),
        out_shape=jax.ShapeDtypeStruct((N_NODES, DIM), jnp.float32),
    )(partials, b2d)


_MESH = plsc.VectorSubcoreMesh(core_axis_name="c", subcore_axis_name="s")


@functools.partial(
    pl.kernel,
    out_type=jax.ShapeDtypeStruct((NUM_CORES, N_PAD, DIM), jnp.float32),
    mesh=_MESH,
    scratch_types=[
        pltpu.VMEM_SHARED((N_PAD, DIM), jnp.float32),
        pltpu.VMEM((NIDX, 4, CHUNK), jnp.int32),
        pltpu.VMEM((NROW, CHUNK, 2 * DIM), jnp.float32),
        pltpu.VMEM((NROW, CHUNK, DIM), jnp.float32),
        pltpu.SemaphoreType.DMA((NIDX,)),
        pltpu.SemaphoreType.DMA((NROW,)),
        pltpu.SemaphoreType.DMA((NROW,)),
    ],
    compiler_params=pltpu.CompilerParams(needs_layout_passes=False),
)
def _sc_aggregate(support_hbm, edges_hbm, out_hbm,
                  acc, idx_v, rows_v, sbuf, isem, gsem, ssem):
    cid = lax.axis_index("c")
    sid = lax.axis_index("s")
    cpw = jnp.where(cid == 0, CPW0, CPW1)
    cbase = jnp.where(cid == 0, sid * CPW0,
                      NUM_SUBCORES * CPW0 + sid * CPW1)
    ngroups = jnp.where(cid == 0, CPW0 // GROUP, CPW1 // GROUP)

    @pl.loop(0, CHUNK)
    def _zero_row(j):
        for k in range(DIM // 16):
            sbuf[0, j, pl.ds(k * 16, 16)] = jnp.zeros((16,), jnp.float32)

    base = sid * ROWS_PER_TILE
    zfull = ROWS_PER_TILE // CHUNK
    zrem = ROWS_PER_TILE - zfull * CHUNK
    for i in range(zfull):
        pltpu.sync_copy(sbuf.at[0], acc.at[pl.ds(base + i * CHUNK, CHUNK)])
    if zrem:
        pltpu.sync_copy(sbuf.at[0, pl.ds(0, zrem)],
                        acc.at[pl.ds(base + zfull * CHUNK, zrem)])

    plsc.subcore_barrier()

    def _idx_copy(j, q):
        return pltpu.async_copy(edges_hbm.at[cbase + j], idx_v.at[q],
                                isem.at[q])

    def _idx_wait(j, q):
        pltpu.make_async_copy(edges_hbm.at[cbase + j], idx_v.at[q],
                              isem.at[q]).wait()

    def _gather(q, b):
        return pltpu.async_copy(support_hbm.at[idx_v.at[q, 0]],
                                rows_v.at[b], gsem.at[b])

    def _gather_wait(q, b):
        pltpu.make_async_copy(support_hbm.at[idx_v.at[q, 0]],
                              rows_v.at[b], gsem.at[b]).wait()

    def _scatter(q, b):
        return pltpu.async_copy(sbuf.at[b], acc.at[idx_v.at[q, 1]],
                                ssem.at[b], add=True)

    def _scatter_wait(q, b):
        pltpu.make_async_copy(sbuf.at[b], acc.at[idx_v.at[q, 1]],
                              ssem.at[b]).wait()

    @pl.when(cpw > 0)
    def _():
        for j in range(2):
            _idx_copy(j, j)
        _idx_wait(0, 0)
        _gather(0, 0)

    @pl.loop(0, ngroups)
    def _grp(g):
        c0 = g * GROUP
        for k in range(GROUP):
            i = c0 + k
            b = k % NROW
            q = k % NIDX
            b1 = (k + 1) % NROW
            q1 = (k + 1) % NIDX
            q2 = (k + 2) % NIDX

            @pl.when(i + 2 < cpw)
            def _():
                _idx_copy(i + 2, q2)

            @pl.when(i + 1 < cpw)
            def _():
                @pl.when(i + 1 >= NROW)
                def _():
                    _scatter_wait(q1, b1)
                _idx_wait(i + 1, q1)
                _gather(q1, b1)

            _gather_wait(q, b)

            @plsc.parallel_loop(0, CHUNK, unroll=2)
            def _edge(j):
                j16 = jnp.full((16,), j, jnp.int32)
                wbits = plsc.load_gather(idx_v.at[q, 2], [j16])
                wj = plsc.bitcast(wbits, jnp.float32)
                par = plsc.load_gather(idx_v.at[q, 3], [j16]) > 0
                for kk in range(DIM // 16):
                    va = rows_v[b, j, pl.ds(kk * 16, 16)]
                    vb = rows_v[b, j, pl.ds(DIM + kk * 16, 16)]
                    sbuf[b, j, pl.ds(kk * 16, 16)] = (
                        jnp.where(par, vb, va) * wj)

            _scatter(q, b)

    @pl.when(cpw > 0)
    def _():
        for k in range(GROUP - NROW, GROUP):
            _scatter_wait(k % NIDX, k % NROW)

    plsc.subcore_barrier()

    for i in range(zfull):
        pltpu.sync_copy(acc.at[pl.ds(base + i * CHUNK, CHUNK)],
                        out_hbm.at[cid, pl.ds(base + i * CHUNK, CHUNK)])
    if zrem:
        pltpu.sync_copy(acc.at[pl.ds(base + zfull * CHUNK, zrem)],
                        out_hbm.at[cid, pl.ds(base + zfull * CHUNK, zrem)])


def kernel(input, edge_index, edge_weight, W, b):
    support = _support_matmul(input, W)

    pad = E_PAD - N_EDGES
    src = jnp.concatenate(
        [edge_index[1].astype(jnp.int32), jnp.zeros((pad,), jnp.int32)])
    dst = jnp.concatenate(
        [edge_index[0].astype(jnp.int32), jnp.zeros((pad,), jnp.int32)])
    wbits = jnp.concatenate(
        [lax.bitcast_convert_type(edge_weight.astype(jnp.float32), jnp.int32),
         jnp.zeros((pad,), jnp.int32)])
    edges = jnp.stack(
        [(src // 2).reshape(TOT_CHUNKS, CHUNK),
         dst.reshape(TOT_CHUNKS, CHUNK),
         wbits.reshape(TOT_CHUNKS, CHUNK),
         (src & 1).reshape(TOT_CHUNKS, CHUNK)], axis=1)

    partials = _sc_aggregate(support.reshape(N_NODES // 2, 2 * DIM), edges)
    return _combine(partials, b.reshape(1, DIM))


# speedup vs baseline: 1.4969x; 1.4969x over previous
import functools

import jax
import jax.numpy as jnp
from jax import lax
from jax.experimental import pallas as pl
from jax.experimental.pallas import tpu as pltpu
from jax.experimental.pallas import tpu_sc as plsc

N_NODES = 10000
N_EDGES = 320000
DIM = 128

NUM_CORES = 2
NUM_SUBCORES = 16
NUM_WORKERS = NUM_CORES * NUM_SUBCORES
CHUNK = 128
GROUP = 8
CPW = 80
E_PAD = NUM_WORKERS * CPW * CHUNK
N_PAD = 10240
ROWS_PER_TILE = N_PAD // NUM_SUBCORES
NROW = 2
NIDX = 8


def _matmul_body(x_ref, w_ref, o_ref):
    o_ref[...] = jnp.dot(x_ref[...], w_ref[...],
                         preferred_element_type=jnp.float32)


def _support_matmul(x, W):
    blk = 400
    grid = N_NODES // blk
    return pl.pallas_call(
        _matmul_body,
        grid=(grid,),
        in_specs=[
            pl.BlockSpec((blk, DIM), lambda i: (i, 0)),
            pl.BlockSpec((DIM, DIM), lambda i: (0, 0)),
        ],
        out_specs=pl.BlockSpec((blk, DIM), lambda i: (i, 0)),
        out_shape=jax.ShapeDtypeStruct((N_NODES, DIM), jnp.float32),
    )(x, W)


def _combine_body(p_ref, b_ref, o_ref):
    o_ref[...] = p_ref[0] + p_ref[1] + b_ref[...]


def _combine(partials, b2d):
    blk = 400
    grid = N_NODES // blk
    return pl.pallas_call(
        _combine_body,
        grid=(grid,),
        in_specs=[
            pl.BlockSpec((2, blk, DIM), lambda i: (0, i, 0)),
            pl.BlockSpec((1, DIM), lambda i: (0, 0)),
        ],
        out_specs=pl.BlockSpec((blk, DIM), lambda i: (i, 0)),
        out_shape=jax.ShapeDtypeStruct((N_NODES, DIM), jnp.float32),
    )(partials, b2d)


_MESH = plsc.VectorSubcoreMesh(core_axis_name="c", subcore_axis_name="s")


@functools.partial(
    pl.kernel,
    out_type=jax.ShapeDtypeStruct((NUM_CORES, N_PAD, DIM), jnp.float32),
    mesh=_MESH,
    scratch_types=[
        pltpu.VMEM_SHARED((N_PAD, DIM), jnp.float32),
        pltpu.VMEM((NIDX, 3, CHUNK), jnp.int32),
        pltpu.VMEM((NROW, CHUNK, DIM), jnp.float32),
        pltpu.SemaphoreType.DMA((NIDX,)),
        pltpu.SemaphoreType.DMA((NROW,)),
        pltpu.SemaphoreType.DMA((NROW,)),
    ],
    compiler_params=pltpu.CompilerParams(needs_layout_passes=False),
)
def _sc_aggregate(support_hbm, edges_hbm, out_hbm,
                  acc, idx_v, rows_v, isem, gsem, ssem):
    cid = lax.axis_index("c")
    sid = lax.axis_index("s")
    wid = cid * NUM_SUBCORES + sid

    @pl.loop(0, CHUNK)
    def _zero_row(j):
        for k in range(DIM // 16):
            rows_v[0, j, pl.ds(k * 16, 16)] = jnp.zeros((16,), jnp.float32)

    base = sid * ROWS_PER_TILE
    for i in range(ROWS_PER_TILE // CHUNK):
        pltpu.sync_copy(rows_v.at[0], acc.at[pl.ds(base + i * CHUNK, CHUNK)])

    plsc.subcore_barrier()

    def _idx_copy(j, q):
        return pltpu.async_copy(edges_hbm.at[wid, j], idx_v.at[q],
                                isem.at[q])

    def _gather(j, q, b):
        del j
        return pltpu.async_copy(support_hbm.at[idx_v.at[q, 0]],
                                rows_v.at[b], gsem.at[b])

    def _scatter(q, b):
        return pltpu.async_copy(rows_v.at[b], acc.at[idx_v.at[q, 1]],
                                ssem.at[b], add=True)

    for j in range(2):
        _idx_copy(j, j)
    pltpu.make_async_copy(edges_hbm.at[wid, 0], idx_v.at[0],
                          isem.at[0]).wait()
    _gather(0, 0, 0)

    @pl.loop(0, CPW // GROUP)
    def _grp(g):
        c0 = g * GROUP
        for k in range(GROUP):
            i = c0 + k
            b = k % NROW
            q = k
            b1 = (k + 1) % NROW
            q1 = (k + 1) % NIDX
            q2 = (k + 2) % NIDX

            @pl.when(i + 2 < CPW)
            def _():
                _idx_copy(i + 2, q2)

            @pl.when(i + 1 < CPW)
            def _():
                @pl.when(i + 1 >= NROW)
                def _():
                    pltpu.make_async_copy(
                        rows_v.at[b1], acc.at[idx_v.at[q1, 1]],
                        ssem.at[b1]).wait()
                pltpu.make_async_copy(edges_hbm.at[wid, i + 1],
                                      idx_v.at[q1], isem.at[q1]).wait()
                _gather(i + 1, q1, b1)

            pltpu.make_async_copy(support_hbm.at[idx_v.at[q, 0]],
                                  rows_v.at[b], gsem.at[b]).wait()

            @plsc.parallel_loop(0, CHUNK, unroll=2)
            def _edge(j):
                wbits = plsc.load_gather(idx_v.at[q, 2],
                                         [jnp.full((16,), j, jnp.int32)])
                wj = plsc.bitcast(wbits, jnp.float32)
                for kk in range(DIM // 16):
                    sl = (b, j, pl.ds(kk * 16, 16))
                    rows_v[sl] = rows_v[sl] * wj

            _scatter(q, b)

    for k in range(GROUP - NROW, GROUP):
        pltpu.make_async_copy(rows_v.at[k % NROW], acc.at[idx_v.at[k, 1]],
                              ssem.at[k % NROW]).wait()

    plsc.subcore_barrier()

    for i in range(ROWS_PER_TILE // CHUNK):
        pltpu.sync_copy(acc.at[pl.ds(base + i * CHUNK, CHUNK)],
                        out_hbm.at[cid, pl.ds(base + i * CHUNK, CHUNK)])


def kernel(input, edge_index, edge_weight, W, b):
    support = _support_matmul(input, W)

    pad = E_PAD - N_EDGES
    src = jnp.concatenate(
        [edge_index[1].astype(jnp.int32), jnp.zeros((pad,), jnp.int32)])
    dst = jnp.concatenate(
        [edge_index[0].astype(jnp.int32), jnp.zeros((pad,), jnp.int32)])
    wbits = jnp.concatenate(
        [lax.bitcast_convert_type(edge_weight.astype(jnp.float32), jnp.int32),
         jnp.zeros((pad,), jnp.int32)])
    edges = jnp.stack(
        [src.reshape(NUM_WORKERS, CPW, CHUNK),
         dst.reshape(NUM_WORKERS, CPW, CHUNK),
         wbits.reshape(NUM_WORKERS, CPW, CHUNK)], axis=2)

    partials = _sc_aggregate(support, edges)
    return _combine(partials, b.reshape(1, DIM))
